# baseline (speedup 1.0000x reference)
import functools

import jax
import jax.numpy as jnp
from jax import lax
from jax.experimental import pallas as pl
from jax.experimental.pallas import tpu as pltpu
from jax.experimental.pallas import tpu_sc as plsc

T = 8192
E = 8
D = 1024
F = 4096
C = 2048

NCORE = 2
NSUB = 16
NW = NCORE * NSUB
L = 16

B = E * C


def _wid():
    return lax.axis_index("s") * NCORE + lax.axis_index("c")


def _last(v):
    return lax.squeeze(lax.slice(v, (L - 1,), (L,)), (0,))


def _mesh():
    return plsc.VectorSubcoreMesh(core_axis_name="c", subcore_axis_name="s")


def _route_body(keys_hbm, idx_hbm, ptr_hbm, wsl_hbm, keys_v, idx_v, ptr_v, wsl_v):
    wid = _wid()

    @pl.when(wid < E)
    def _():
        pltpu.sync_copy(keys_hbm.at[wid], keys_v)

        def zero_idx(i, carry):
            idx_v[pl.ds(i * L, L)] = jnp.zeros((L,), jnp.int32)
            wsl_v[pl.ds(i * L, L)] = jnp.zeros((L,), jnp.float32)
            return carry

        lax.fori_loop(0, (C + L) // L, zero_idx, 0, unroll=4)

        def count_ge(v):
            def body(i, acc):
                k = keys_v[pl.ds(i * L, L)]
                return acc + jnp.where(k >= v, 1, 0).astype(jnp.int32)

            return jnp.sum(lax.fori_loop(0, T // L, body,
                                         jnp.zeros((L,), jnp.int32), unroll=8))

        n_masked = count_ge(jnp.int32(0))
        nsel = jnp.minimum(n_masked, jnp.int32(C))

        def bs_body(_, lohi):
            lo, hi = lohi
            mid = (lo + hi) >> 1
            ge = count_ge(mid)
            big = ge >= nsel
            return jnp.where(big, mid, lo), jnp.where(big, hi, mid)

        lo, _ = lax.fori_loop(0, 31, bs_body, (jnp.int32(-1), jnp.int32(1 << 30)))
        n_gt = count_ge(lo + 1)
        has = nsel > 0
        thr = jnp.where(has, lo, jnp.int32(0x7FFFFFFF))
        quota = jnp.where(has, nsel - n_gt, jnp.int32(0))

        def compact(i, carry):
            s, c = carry
            k = keys_v[pl.ds(i * L, L)]
            gt = k > thr
            eq = k == thr
            eqc = plsc.cumsum(jnp.where(eq, 1, 0).astype(jnp.int32))
            sel = gt | (eq & ((c + eqc) <= quota))
            selc = plsc.cumsum(jnp.where(sel, 1, 0).astype(jnp.int32))
            slot = (s + selc) - 1
            ptr_v[pl.ds(i * L, L)] = jnp.where(sel, wid * C + slot, B)
            tok = lax.iota(jnp.int32, L) + i * L
            plsc.store_compressed(idx_v.at[pl.ds(s, L)], tok, mask=sel)
            plsc.store_compressed(wsl_v.at[pl.ds(s, L)],
                                  plsc.bitcast(k, jnp.float32), mask=sel)
            return s + _last(selc), c + _last(eqc)

        lax.fori_loop(0, T // L, compact, (jnp.int32(0), jnp.int32(0)))

        pltpu.sync_copy(idx_v.at[pl.ds(0, C)], idx_hbm.at[wid])
        pltpu.sync_copy(ptr_v, ptr_hbm.at[wid])
        pltpu.sync_copy(wsl_v.at[pl.ds(0, C)], wsl_hbm.at[wid])


def _route(keys):
    return pl.kernel(
        _route_body,
        out_type=(
            jax.ShapeDtypeStruct((E, C), jnp.int32),
            jax.ShapeDtypeStruct((E, T), jnp.int32),
            jax.ShapeDtypeStruct((E, C), jnp.float32),
        ),
        mesh=_mesh(),
        scratch_types=[
            pltpu.VMEM((T,), jnp.int32),
            pltpu.VMEM((C + L,), jnp.int32),
            pltpu.VMEM((T,), jnp.int32),
            pltpu.VMEM((C + L,), jnp.float32),
        ],
        compiler_params=pltpu.CompilerParams(needs_layout_passes=False),
    )(keys)


_GR = 32
_GB = B // NW
_GN = _GB // _GR


def _gather_body(x_hbm, idxf_hbm, xg_hbm, idx_v, buf0, buf1, sem0, sem1):
    wid = _wid()
    base = wid * _GB
    pltpu.sync_copy(idxf_hbm.at[pl.ds(base, _GB)], idx_v)
    bufs = (buf0, buf1)
    sems = (sem0, sem1)
    pend = [None, None]
    pend[0] = pltpu.async_copy(x_hbm.at[idx_v.at[pl.ds(0, _GR)]], buf0, sem0)
    for j in range(_GN):
        if j + 1 < _GN:
            pend[(j + 1) % 2] = pltpu.async_copy(
                x_hbm.at[idx_v.at[pl.ds((j + 1) * _GR, _GR)]],
                bufs[(j + 1) % 2], sems[(j + 1) % 2])
        pend[j % 2].wait()
        pltpu.sync_copy(bufs[j % 2], xg_hbm.at[pl.ds(base + j * _GR, _GR)])


def _gather(x, idx_flat):
    return pl.kernel(
        _gather_body,
        out_type=jax.ShapeDtypeStruct((B, D), jnp.float32),
        mesh=_mesh(),
        scratch_types=[
            pltpu.VMEM((_GB,), jnp.int32),
            pltpu.VMEM((_GR, D), jnp.float32),
            pltpu.VMEM((_GR, D), jnp.float32),
            pltpu.SemaphoreType.DMA,
            pltpu.SemaphoreType.DMA,
        ],
    )(x, idx_flat)


_BC = 256
_BF = 2048
_NF = F // _BF


def _mlp_body(xg_ref, w1_ref, b1_ref, w2_ref, b2_ref, ws_ref, out_ref, acc_ref):
    f = pl.program_id(1)
    c = pl.program_id(2)
    xb = xg_ref[0].astype(jnp.bfloat16)
    h = jnp.dot(xb, w1_ref[0].astype(jnp.bfloat16),
                preferred_element_type=jnp.float32) + b1_ref[0, 0]
    h = jax.nn.gelu(h)
    part = jnp.dot(h.astype(jnp.bfloat16), w2_ref[0].astype(jnp.bfloat16),
                   preferred_element_type=jnp.float32)
    sl = pl.ds(c * _BC, _BC)

    @pl.when(f == 0)
    def _():
        acc_ref[sl, :] = part

    @pl.when(f == _NF - 1)
    def _():
        out_ref[0] = ((acc_ref[sl, :] + part + b2_ref[0, 0][None, :])
                      * ws_ref[0, 0][:, None])


def _mlp(xg, W1, b1, W2, b2, wslot):
    return pl.pallas_call(
        _mlp_body,
        grid=(E, _NF, C // _BC),
        in_specs=[
            pl.BlockSpec((1, _BC, D), lambda e, f, c: (e, c, 0)),
            pl.BlockSpec((1, D, _BF), lambda e, f, c: (e, 0, f)),
            pl.BlockSpec((1, 1, _BF), lambda e, f, c: (e * _NF + f, 0, 0)),
            pl.BlockSpec((1, _BF, D), lambda e, f, c: (e, f, 0)),
            pl.BlockSpec((1, 1, D), lambda e, f, c: (e, 0, 0)),
            pl.BlockSpec((1, 1, _BC), lambda e, f, c: (e * (C // _BC) + c, 0, 0)),
        ],
        out_specs=pl.BlockSpec((1, _BC, D), lambda e, f, c: (e, c, 0)),
        out_shape=jax.ShapeDtypeStruct((E, C, D), jnp.float32),
        scratch_shapes=[pltpu.VMEM((C, D), jnp.float32)],
        compiler_params=pltpu.CompilerParams(
            dimension_semantics=("arbitrary", "arbitrary", "arbitrary")),
    )(xg, W1, b1.reshape(E * _NF, 1, _BF), W2, b2.reshape(E, 1, D),
      wslot.reshape(E * (C // _BC), 1, _BC))


_RT = 32
_TB = T // NW


_PCAP = E * _RT + 2 * L


def _combine_body(outbuf_hbm, ptr_hbm, y_hbm, ptrs_v, plist, tgte, gbuf, acc_v, sem):
    wid = _wid()
    t0 = wid * _TB
    for e in range(E):
        pltpu.sync_copy(ptr_hbm.at[e, pl.ds(t0, _TB)], ptrs_v.at[e])
    iota = lax.iota(jnp.int32, L)
    padfill = jnp.int32(B) + (iota & 7)

    def chunk(j, carry):
        def prefill(i, c2):
            plist[pl.ds(i * L, L)] = padfill
            return c2

        lax.fori_loop(0, _PCAP // L, prefill, 0, unroll=4)

        def pretgt(i, c2):
            tgte[pl.ds(i * L, L)] = jnp.full((L,), _RT, jnp.int32)
            return c2

        lax.fori_loop(0, _PCAP, pretgt, 0, unroll=8)

        def zacc(i, c2):
            def zcol(v, c3):
                acc_v[i, pl.ds(v * L, L)] = jnp.zeros((L,), jnp.float32)
                return c3

            return lax.fori_loop(0, D // L, zcol, c2, unroll=8)

        lax.fori_loop(0, _RT, zacc, 0)

        n = jnp.int32(0)
        for e in range(E):
            for g in range(_RT // L):
                p = ptrs_v[e, pl.ds(j * _RT + g * L, L)]
                sel = p != B
                cnt = plsc.cumsum(jnp.where(sel, 1, 0).astype(jnp.int32))
                excl = cnt - jnp.where(sel, 1, 0).astype(jnp.int32)
                plsc.store_compressed(plist.at[pl.ds(n, L)], p, mask=sel)
                plsc.store_scatter(tgte, [(n + excl) * L], g * L + iota, mask=sel)
                n = n + _last(cnt)
        nch = (n + (_RT - 1)) >> 5

        def kchunk(k, c2):
            pltpu.async_copy(
                outbuf_hbm.at[plist.at[pl.ds(k * _RT, _RT)]], gbuf, sem).wait()

            def row(i, c3):
                tv = tgte[pl.ds((k * _RT + i) * L, L)]
                tgt = lax.squeeze(lax.slice(tv, (0,), (1,)), (0,))
                for vg in range(D // L // 16):
                    xs = [gbuf[i, pl.ds((vg * 16 + v) * L, L)] for v in range(16)]
                    for v in range(16):
                        plsc.addupdate(acc_v.at[tgt, pl.ds((vg * 16 + v) * L, L)],
                                       xs[v])
                return c3

            return lax.fori_loop(0, _RT, row, c2)

        lax.fori_loop(0, nch, kchunk, 0)
        pltpu.sync_copy(acc_v.at[pl.ds(0, _RT)], y_hbm.at[pl.ds(t0 + j * _RT, _RT)])
        return carry

    lax.fori_loop(0, _TB // _RT, chunk, 0)


def _combine(outbuf, ptr):
    return pl.kernel(
        _combine_body,
        out_type=jax.ShapeDtypeStruct((T, D), jnp.float32),
        mesh=_mesh(),
        scratch_types=[
            pltpu.VMEM((E, _TB), jnp.int32),
            pltpu.VMEM((_PCAP,), jnp.int32),
            pltpu.VMEM((_PCAP * L,), jnp.int32),
            pltpu.VMEM((_RT, D), jnp.float32),
            pltpu.VMEM((_RT + 1, D), jnp.float32),
            pltpu.SemaphoreType.DMA,
        ],
        compiler_params=pltpu.CompilerParams(needs_layout_passes=False),
    )(outbuf, ptr)


def kernel(x, route_mask, route_weight, W1, b1, W2, b2):
    keys = jnp.where(route_mask, lax.bitcast_convert_type(route_weight, jnp.int32),
                     jnp.int32(-1)).T
    idx, ptr, wslot = _route(keys)
    xg = _gather(x, idx.reshape(B))
    out = _mlp(xg.reshape(E, C, D), W1, b1, W2, b2, wslot)
    outbuf = jnp.concatenate([out.reshape(B, D), jnp.zeros((8, D), jnp.float32)], 0)
    y = _combine(outbuf, ptr)
    return y


# speedup vs baseline: 1.0508x; 1.0508x over previous
import functools

import jax
import jax.numpy as jnp
from jax import lax
from jax.experimental import pallas as pl
from jax.experimental.pallas import tpu as pltpu
from jax.experimental.pallas import tpu_sc as plsc

T = 8192
E = 8
D = 1024
F = 4096
C = 2048

NCORE = 2
NSUB = 16
NW = NCORE * NSUB
L = 16

B = E * C


def _wid():
    return lax.axis_index("s") * NCORE + lax.axis_index("c")


def _last(v):
    return lax.squeeze(lax.slice(v, (L - 1,), (L,)), (0,))


def _mesh():
    return plsc.VectorSubcoreMesh(core_axis_name="c", subcore_axis_name="s")


def _route_body(keys_hbm, idx_hbm, ptr_hbm, wsl_hbm, keys_v, idx_v, ptr_v, wsl_v):
    wid = _wid()

    @pl.when(wid < E)
    def _():
        pltpu.sync_copy(keys_hbm.at[wid], keys_v)

        def zero_idx(i, carry):
            idx_v[pl.ds(i * L, L)] = jnp.zeros((L,), jnp.int32)
            wsl_v[pl.ds(i * L, L)] = jnp.zeros((L,), jnp.float32)
            return carry

        lax.fori_loop(0, (C + L) // L, zero_idx, 0, unroll=4)

        def count_ge(v):
            def body(i, acc):
                k = keys_v[pl.ds(i * L, L)]
                return acc + jnp.where(k >= v, 1, 0).astype(jnp.int32)

            return jnp.sum(lax.fori_loop(0, T // L, body,
                                         jnp.zeros((L,), jnp.int32), unroll=8))

        n_masked = count_ge(jnp.int32(0))
        nsel = jnp.minimum(n_masked, jnp.int32(C))

        def bs_body(_, lohi):
            lo, hi = lohi
            mid = (lo + hi) >> 1
            ge = count_ge(mid)
            big = ge >= nsel
            return jnp.where(big, mid, lo), jnp.where(big, hi, mid)

        lo, _ = lax.fori_loop(0, 31, bs_body, (jnp.int32(-1), jnp.int32(1 << 30)))
        n_gt = count_ge(lo + 1)
        has = nsel > 0
        thr = jnp.where(has, lo, jnp.int32(0x7FFFFFFF))
        quota = jnp.where(has, nsel - n_gt, jnp.int32(0))

        def compact(i, carry):
            s, c = carry
            k = keys_v[pl.ds(i * L, L)]
            gt = k > thr
            eq = k == thr
            eqc = plsc.cumsum(jnp.where(eq, 1, 0).astype(jnp.int32))
            sel = gt | (eq & ((c + eqc) <= quota))
            selc = plsc.cumsum(jnp.where(sel, 1, 0).astype(jnp.int32))
            slot = (s + selc) - 1
            ptr_v[pl.ds(i * L, L)] = jnp.where(sel, wid * C + slot, B)
            tok = lax.iota(jnp.int32, L) + i * L
            plsc.store_compressed(idx_v.at[pl.ds(s, L)], tok, mask=sel)
            plsc.store_compressed(wsl_v.at[pl.ds(s, L)],
                                  plsc.bitcast(k, jnp.float32), mask=sel)
            return s + _last(selc), c + _last(eqc)

        lax.fori_loop(0, T // L, compact, (jnp.int32(0), jnp.int32(0)))

        pltpu.sync_copy(idx_v.at[pl.ds(0, C)], idx_hbm.at[wid])
        pltpu.sync_copy(ptr_v, ptr_hbm.at[wid])
        pltpu.sync_copy(wsl_v.at[pl.ds(0, C)], wsl_hbm.at[wid])


def _route(keys):
    return pl.kernel(
        _route_body,
        out_type=(
            jax.ShapeDtypeStruct((E, C), jnp.int32),
            jax.ShapeDtypeStruct((E, T), jnp.int32),
            jax.ShapeDtypeStruct((E, C), jnp.float32),
        ),
        mesh=_mesh(),
        scratch_types=[
            pltpu.VMEM((T,), jnp.int32),
            pltpu.VMEM((C + L,), jnp.int32),
            pltpu.VMEM((T,), jnp.int32),
            pltpu.VMEM((C + L,), jnp.float32),
        ],
        compiler_params=pltpu.CompilerParams(needs_layout_passes=False),
    )(keys)


_GR = 32
_GB = B // NW
_GN = _GB // _GR


def _gather_body(x_hbm, idxf_hbm, xg_hbm, idx_v, buf0, buf1, sem0, sem1):
    wid = _wid()
    base = wid * _GB
    pltpu.sync_copy(idxf_hbm.at[pl.ds(base, _GB)], idx_v)
    bufs = (buf0, buf1)
    sems = (sem0, sem1)
    pend = [None, None]
    pend[0] = pltpu.async_copy(x_hbm.at[idx_v.at[pl.ds(0, _GR)]], buf0, sem0)
    for j in range(_GN):
        if j + 1 < _GN:
            pend[(j + 1) % 2] = pltpu.async_copy(
                x_hbm.at[idx_v.at[pl.ds((j + 1) * _GR, _GR)]],
                bufs[(j + 1) % 2], sems[(j + 1) % 2])
        pend[j % 2].wait()
        pltpu.sync_copy(bufs[j % 2], xg_hbm.at[pl.ds(base + j * _GR, _GR)])


def _gather(x, idx_flat):
    return pl.kernel(
        _gather_body,
        out_type=jax.ShapeDtypeStruct((B, D), jnp.float32),
        mesh=_mesh(),
        scratch_types=[
            pltpu.VMEM((_GB,), jnp.int32),
            pltpu.VMEM((_GR, D), jnp.float32),
            pltpu.VMEM((_GR, D), jnp.float32),
            pltpu.SemaphoreType.DMA,
            pltpu.SemaphoreType.DMA,
        ],
    )(x, idx_flat)


_BC = 256
_BF = 2048
_NF = F // _BF


def _mlp_body(xg_ref, w1_ref, b1_ref, w2_ref, b2_ref, ws_ref, out_ref, acc_ref):
    f = pl.program_id(1)
    c = pl.program_id(2)
    xb = xg_ref[0]
    h = jnp.dot(xb, w1_ref[0], preferred_element_type=jnp.float32) + b1_ref[0, 0]
    h = jax.nn.gelu(h.astype(jnp.bfloat16)).astype(jnp.float32)
    part = jnp.dot(h, w2_ref[0], preferred_element_type=jnp.float32)
    sl = pl.ds(c * _BC, _BC)

    @pl.when(f == 0)
    def _():
        acc_ref[sl, :] = part

    @pl.when(f == _NF - 1)
    def _():
        out_ref[0] = ((acc_ref[sl, :] + part + b2_ref[0, 0][None, :])
                      * ws_ref[0, 0][:, None])


def _mlp(xg, W1, b1, W2, b2, wslot):
    return pl.pallas_call(
        _mlp_body,
        grid=(E, _NF, C // _BC),
        in_specs=[
            pl.BlockSpec((1, _BC, D), lambda e, f, c: (e, c, 0)),
            pl.BlockSpec((1, D, _BF), lambda e, f, c: (e, 0, f)),
            pl.BlockSpec((1, 1, _BF), lambda e, f, c: (e * _NF + f, 0, 0)),
            pl.BlockSpec((1, _BF, D), lambda e, f, c: (e, f, 0)),
            pl.BlockSpec((1, 1, D), lambda e, f, c: (e, 0, 0)),
            pl.BlockSpec((1, 1, _BC), lambda e, f, c: (e * (C // _BC) + c, 0, 0)),
        ],
        out_specs=pl.BlockSpec((1, _BC, D), lambda e, f, c: (e, c, 0)),
        out_shape=jax.ShapeDtypeStruct((E, C, D), jnp.float32),
        scratch_shapes=[pltpu.VMEM((C, D), jnp.float32)],
        compiler_params=pltpu.CompilerParams(
            dimension_semantics=("arbitrary", "arbitrary", "arbitrary")),
    )(xg, W1, b1.reshape(E * _NF, 1, _BF), W2, b2.reshape(E, 1, D),
      wslot.reshape(E * (C // _BC), 1, _BC))


_RT = 32
_TB = T // NW


_PCAP = E * _RT + 2 * L


def _combine_body(outbuf_hbm, ptr_hbm, y_hbm, ptrs_v, plist, tgte, gbuf, acc_v, sem):
    wid = _wid()
    t0 = wid * _TB
    for e in range(E):
        pltpu.sync_copy(ptr_hbm.at[e, pl.ds(t0, _TB)], ptrs_v.at[e])
    iota = lax.iota(jnp.int32, L)
    padfill = jnp.int32(B) + (iota & 7)

    def chunk(j, carry):
        def prefill(i, c2):
            plist[pl.ds(i * L, L)] = padfill
            return c2

        lax.fori_loop(0, _PCAP // L, prefill, 0, unroll=4)

        def pretgt(i, c2):
            tgte[pl.ds(i * L, L)] = jnp.full((L,), _RT, jnp.int32)
            return c2

        lax.fori_loop(0, _PCAP, pretgt, 0, unroll=8)

        def zacc(i, c2):
            def zcol(v, c3):
                acc_v[i, pl.ds(v * L, L)] = jnp.zeros((L,), jnp.float32)
                return c3

            return lax.fori_loop(0, D // L, zcol, c2, unroll=8)

        lax.fori_loop(0, _RT, zacc, 0)

        n = jnp.int32(0)
        for e in range(E):
            for g in range(_RT // L):
                p = ptrs_v[e, pl.ds(j * _RT + g * L, L)]
                sel = p != B
                cnt = plsc.cumsum(jnp.where(sel, 1, 0).astype(jnp.int32))
                excl = cnt - jnp.where(sel, 1, 0).astype(jnp.int32)
                plsc.store_compressed(plist.at[pl.ds(n, L)], p, mask=sel)
                plsc.store_scatter(tgte, [(n + excl) * L], g * L + iota, mask=sel)
                n = n + _last(cnt)
        nch = (n + (_RT - 1)) >> 5

        def kchunk(k, c2):
            pltpu.async_copy(
                outbuf_hbm.at[plist.at[pl.ds(k * _RT, _RT)]], gbuf, sem).wait()

            def row(i, c3):
                tv = tgte[pl.ds((k * _RT + i) * L, L)]
                tgt = lax.squeeze(lax.slice(tv, (0,), (1,)), (0,))
                for vg in range(D // L // 16):
                    xs = [gbuf[i, pl.ds((vg * 16 + v) * L, L)] for v in range(16)]
                    for v in range(16):
                        plsc.addupdate(acc_v.at[tgt, pl.ds((vg * 16 + v) * L, L)],
                                       xs[v])
                return c3

            return lax.fori_loop(0, _RT, row, c2)

        lax.fori_loop(0, nch, kchunk, 0)
        pltpu.sync_copy(acc_v.at[pl.ds(0, _RT)], y_hbm.at[pl.ds(t0 + j * _RT, _RT)])
        return carry

    lax.fori_loop(0, _TB // _RT, chunk, 0)


def _combine(outbuf, ptr):
    return pl.kernel(
        _combine_body,
        out_type=jax.ShapeDtypeStruct((T, D), jnp.float32),
        mesh=_mesh(),
        scratch_types=[
            pltpu.VMEM((E, _TB), jnp.int32),
            pltpu.VMEM((_PCAP,), jnp.int32),
            pltpu.VMEM((_PCAP * L,), jnp.int32),
            pltpu.VMEM((_RT, D), jnp.float32),
            pltpu.VMEM((_RT + 1, D), jnp.float32),
            pltpu.SemaphoreType.DMA,
        ],
        compiler_params=pltpu.CompilerParams(needs_layout_passes=False),
    )(outbuf, ptr)


def kernel(x, route_mask, route_weight, W1, b1, W2, b2):
    keys = jnp.where(route_mask, lax.bitcast_convert_type(route_weight, jnp.int32),
                     jnp.int32(-1)).T
    idx, ptr, wslot = _route(keys)
    xg = _gather(x, idx.reshape(B))
    out = _mlp(xg.reshape(E, C, D), W1, b1, W2, b2, wslot)
    outbuf = jnp.concatenate([out.reshape(B, D), jnp.zeros((8, D), jnp.float32)], 0)
    y = _combine(outbuf, ptr)
    return y


# speedup vs baseline: 1.1371x; 1.0821x over previous
import functools

import jax
import jax.numpy as jnp
from jax import lax
from jax.experimental import pallas as pl
from jax.experimental.pallas import tpu as pltpu
from jax.experimental.pallas import tpu_sc as plsc

T = 8192
E = 8
D = 1024
F = 4096
C = 2048

NCORE = 2
NSUB = 16
NW = NCORE * NSUB
L = 16

B = E * C


def _wid():
    return lax.axis_index("s") * NCORE + lax.axis_index("c")


def _last(v):
    return lax.squeeze(lax.slice(v, (L - 1,), (L,)), (0,))


def _mesh():
    return plsc.VectorSubcoreMesh(core_axis_name="c", subcore_axis_name="s")


def _route_body(keys_hbm, idx_hbm, ptr_hbm, wsl_hbm, keys_v, idx_v, ptr_v, wsl_v):
    wid = _wid()

    @pl.when(wid < E)
    def _():
        pltpu.sync_copy(keys_hbm.at[wid], keys_v)

        def zero_idx(i, carry):
            idx_v[pl.ds(i * L, L)] = jnp.zeros((L,), jnp.int32)
            wsl_v[pl.ds(i * L, L)] = jnp.zeros((L,), jnp.float32)
            return carry

        lax.fori_loop(0, (C + L) // L, zero_idx, 0, unroll=4)

        def count_ge(v):
            def body(i, acc):
                k = keys_v[pl.ds(i * L, L)]
                return acc + jnp.where(k >= v, 1, 0).astype(jnp.int32)

            return jnp.sum(lax.fori_loop(0, T // L, body,
                                         jnp.zeros((L,), jnp.int32), unroll=8))

        n_masked = count_ge(jnp.int32(0))
        nsel = jnp.minimum(n_masked, jnp.int32(C))

        def bs_body(_, lohi):
            lo, hi = lohi
            mid = (lo + hi) >> 1
            ge = count_ge(mid)
            big = ge >= nsel
            return jnp.where(big, mid, lo), jnp.where(big, hi, mid)

        lo, _ = lax.fori_loop(0, 31, bs_body, (jnp.int32(-1), jnp.int32(1 << 30)))
        n_gt = count_ge(lo + 1)
        has = nsel > 0
        thr = jnp.where(has, lo, jnp.int32(0x7FFFFFFF))
        quota = jnp.where(has, nsel - n_gt, jnp.int32(0))

        def compact(i, carry):
            s, c = carry
            k = keys_v[pl.ds(i * L, L)]
            gt = k > thr
            eq = k == thr
            eqc = plsc.cumsum(jnp.where(eq, 1, 0).astype(jnp.int32))
            sel = gt | (eq & ((c + eqc) <= quota))
            selc = plsc.cumsum(jnp.where(sel, 1, 0).astype(jnp.int32))
            slot = (s + selc) - 1
            ptr_v[pl.ds(i * L, L)] = jnp.where(sel, wid * C + slot, B)
            tok = lax.iota(jnp.int32, L) + i * L
            plsc.store_compressed(idx_v.at[pl.ds(s, L)], tok, mask=sel)
            plsc.store_compressed(wsl_v.at[pl.ds(s, L)],
                                  plsc.bitcast(k, jnp.float32), mask=sel)
            return s + _last(selc), c + _last(eqc)

        lax.fori_loop(0, T // L, compact, (jnp.int32(0), jnp.int32(0)))

        pltpu.sync_copy(idx_v.at[pl.ds(0, C)], idx_hbm.at[wid])
        pltpu.sync_copy(ptr_v, ptr_hbm.at[wid])
        pltpu.sync_copy(wsl_v.at[pl.ds(0, C)], wsl_hbm.at[wid])


def _route(keys):
    return pl.kernel(
        _route_body,
        out_type=(
            jax.ShapeDtypeStruct((E, C), jnp.int32),
            jax.ShapeDtypeStruct((E, T), jnp.int32),
            jax.ShapeDtypeStruct((E, C), jnp.float32),
        ),
        mesh=_mesh(),
        scratch_types=[
            pltpu.VMEM((T,), jnp.int32),
            pltpu.VMEM((C + L,), jnp.int32),
            pltpu.VMEM((T,), jnp.int32),
            pltpu.VMEM((C + L,), jnp.float32),
        ],
        compiler_params=pltpu.CompilerParams(needs_layout_passes=False),
    )(keys)


_GR = 32
_GB = B // NW
_GN = _GB // _GR


def _gather_body(x_hbm, idxf_hbm, xg_hbm, idx_v, buf0, buf1, sem0, sem1):
    wid = _wid()
    base = wid * _GB
    pltpu.sync_copy(idxf_hbm.at[pl.ds(base, _GB)], idx_v)
    bufs = (buf0, buf1)
    sems = (sem0, sem1)
    pend = [None, None]
    pend[0] = pltpu.async_copy(x_hbm.at[idx_v.at[pl.ds(0, _GR)]], buf0, sem0)
    for j in range(_GN):
        if j + 1 < _GN:
            pend[(j + 1) % 2] = pltpu.async_copy(
                x_hbm.at[idx_v.at[pl.ds((j + 1) * _GR, _GR)]],
                bufs[(j + 1) % 2], sems[(j + 1) % 2])
        pend[j % 2].wait()
        pltpu.sync_copy(bufs[j % 2], xg_hbm.at[pl.ds(base + j * _GR, _GR)])


def _gather(x, idx_flat):
    return pl.kernel(
        _gather_body,
        out_type=jax.ShapeDtypeStruct((B, D), jnp.float32),
        mesh=_mesh(),
        scratch_types=[
            pltpu.VMEM((_GB,), jnp.int32),
            pltpu.VMEM((_GR, D), jnp.float32),
            pltpu.VMEM((_GR, D), jnp.float32),
            pltpu.SemaphoreType.DMA,
            pltpu.SemaphoreType.DMA,
        ],
    )(x, idx_flat)


_BC = 512
_BF = 2048
_NF = F // _BF


def _mlp_body(xg_ref, w1_ref, b1_ref, w2_ref, b2_ref, ws_ref, out_ref, acc_ref):
    f = pl.program_id(1)
    c = pl.program_id(2)
    xb = xg_ref[0]
    h = jnp.dot(xb, w1_ref[0], preferred_element_type=jnp.float32) + b1_ref[0, 0]
    h = jax.nn.gelu(h.astype(jnp.bfloat16)).astype(jnp.float32)
    part = jnp.dot(h, w2_ref[0], preferred_element_type=jnp.float32)
    sl = pl.ds(c * _BC, _BC)

    @pl.when(f == 0)
    def _():
        acc_ref[sl, :] = part

    @pl.when(f == _NF - 1)
    def _():
        out_ref[0] = ((acc_ref[sl, :] + part + b2_ref[0, 0][None, :])
                      * ws_ref[0, 0][:, None])


def _mlp(xg, W1, b1, W2, b2, wslot):
    return pl.pallas_call(
        _mlp_body,
        grid=(E, _NF, C // _BC),
        in_specs=[
            pl.BlockSpec((1, _BC, D), lambda e, f, c: (e, c, 0)),
            pl.BlockSpec((1, D, _BF), lambda e, f, c: (e, 0, f)),
            pl.BlockSpec((1, 1, _BF), lambda e, f, c: (e * _NF + f, 0, 0)),
            pl.BlockSpec((1, _BF, D), lambda e, f, c: (e, f, 0)),
            pl.BlockSpec((1, 1, D), lambda e, f, c: (e, 0, 0)),
            pl.BlockSpec((1, 1, _BC), lambda e, f, c: (e * (C // _BC) + c, 0, 0)),
        ],
        out_specs=pl.BlockSpec((1, _BC, D), lambda e, f, c: (e, c, 0)),
        out_shape=jax.ShapeDtypeStruct((E, C, D), jnp.float32),
        scratch_shapes=[pltpu.VMEM((C, D), jnp.float32)],
        compiler_params=pltpu.CompilerParams(
            dimension_semantics=("arbitrary", "arbitrary", "arbitrary")),
    )(xg, W1, b1.reshape(E * _NF, 1, _BF), W2, b2.reshape(E, 1, D),
      wslot.reshape(E * (C // _BC), 1, _BC))


_RT = 32
_TB = T // NW


_PCAP = E * _RT + 2 * L


def _combine_body(outbuf_hbm, ptr_hbm, y_hbm, ptrs_v, plist, tgte, gbuf, acc_v, sem):
    wid = _wid()
    t0 = wid * _TB
    for e in range(E):
        pltpu.sync_copy(ptr_hbm.at[e, pl.ds(t0, _TB)], ptrs_v.at[e])
    iota = lax.iota(jnp.int32, L)
    padfill = jnp.int32(B) + (iota & 7)

    def chunk(j, carry):
        def prefill(i, c2):
            plist[pl.ds(i * L, L)] = padfill
            return c2

        lax.fori_loop(0, _PCAP // L, prefill, 0, unroll=4)

        def pretgt(i, c2):
            tgte[pl.ds(i * L, L)] = jnp.full((L,), _RT, jnp.int32)
            return c2

        lax.fori_loop(0, _PCAP, pretgt, 0, unroll=8)

        def zacc(i, c2):
            def zcol(v, c3):
                acc_v[i, pl.ds(v * L, L)] = jnp.zeros((L,), jnp.float32)
                return c3

            return lax.fori_loop(0, D // L, zcol, c2, unroll=8)

        lax.fori_loop(0, _RT, zacc, 0)

        n = jnp.int32(0)
        for e in range(E):
            for g in range(_RT // L):
                p = ptrs_v[e, pl.ds(j * _RT + g * L, L)]
                sel = p != B
                cnt = plsc.cumsum(jnp.where(sel, 1, 0).astype(jnp.int32))
                excl = cnt - jnp.where(sel, 1, 0).astype(jnp.int32)
                plsc.store_compressed(plist.at[pl.ds(n, L)], p, mask=sel)
                plsc.store_scatter(tgte, [(n + excl) * L], g * L + iota, mask=sel)
                n = n + _last(cnt)
        nch = (n + (_RT - 1)) >> 5

        def kchunk(k, c2):
            pltpu.async_copy(
                outbuf_hbm.at[plist.at[pl.ds(k * _RT, _RT)]], gbuf, sem).wait()

            def row(i, c3):
                tv = tgte[pl.ds((k * _RT + i) * L, L)]
                tgt = lax.squeeze(lax.slice(tv, (0,), (1,)), (0,))
                for vg in range(D // L // 16):
                    xs = [gbuf[i, pl.ds((vg * 16 + v) * L, L)] for v in range(16)]
                    for v in range(16):
                        plsc.addupdate(acc_v.at[tgt, pl.ds((vg * 16 + v) * L, L)],
                                       xs[v])
                return c3

            return lax.fori_loop(0, _RT, row, c2)

        lax.fori_loop(0, nch, kchunk, 0)
        pltpu.sync_copy(acc_v.at[pl.ds(0, _RT)], y_hbm.at[pl.ds(t0 + j * _RT, _RT)])
        return carry

    lax.fori_loop(0, _TB // _RT, chunk, 0)


def _combine(outbuf, ptr):
    return pl.kernel(
        _combine_body,
        out_type=jax.ShapeDtypeStruct((T, D), jnp.float32),
        mesh=_mesh(),
        scratch_types=[
            pltpu.VMEM((E, _TB), jnp.int32),
            pltpu.VMEM((_PCAP,), jnp.int32),
            pltpu.VMEM((_PCAP * L,), jnp.int32),
            pltpu.VMEM((_RT, D), jnp.float32),
            pltpu.VMEM((_RT + 1, D), jnp.float32),
            pltpu.SemaphoreType.DMA,
        ],
        compiler_params=pltpu.CompilerParams(needs_layout_passes=False),
    )(outbuf, ptr)


def kernel(x, route_mask, route_weight, W1, b1, W2, b2):
    keys = jnp.where(route_mask, lax.bitcast_convert_type(route_weight, jnp.int32),
                     jnp.int32(-1)).T
    idx, ptr, wslot = _route(keys)
    xg = _gather(x, idx.reshape(B))
    out = _mlp(xg.reshape(E, C, D), W1, b1, W2, b2, wslot)
    outbuf = jnp.concatenate([out.reshape(B, D), jnp.zeros((8, D), jnp.float32)], 0)
    y = _combine(outbuf, ptr)
    return y
